# Initial kernel scaffold; baseline (speedup 1.0000x reference)
import jax
import jax.numpy as jnp
from jax.experimental import pallas as pl


def kernel(node_features, batch, graph_attr, W1, b1, gamma, beta, W2, b2):
    raise NotImplementedError("write your pallas kernel here")


# speedup vs baseline: 4.5436x; 4.5436x over previous
import functools

import jax
import jax.numpy as jnp
from jax import lax
from jax.experimental import pallas as pl
from jax.experimental.pallas import tpu as pltpu
from jax.experimental.pallas import tpu_sc as plsc

_G = 512
_N = 100000
_D = 128
_BLK = 128
_NB_FULL = _N // _BLK
_TAIL = _N - _NB_FULL * _BLK
_NW = 32
_CW = 128
_RPT = _G // 16

_BASE = _NB_FULL // _NW
_REM = _NB_FULL - _BASE * _NW


def _segment_sums(feat, ids, zrows, ones_rows):
    mesh = plsc.VectorSubcoreMesh(core_axis_name="c", subcore_axis_name="s")

    @functools.partial(
        pl.kernel,
        out_type=[
            jax.ShapeDtypeStruct((2 * _G, _D), jnp.float32),
            jax.ShapeDtypeStruct((2 * _G, _CW), jnp.float32),
        ],
        mesh=mesh,
        scratch_types=[
            pltpu.VMEM((_BLK, _D), jnp.float32),
            pltpu.VMEM((_BLK,), jnp.int32),
            pltpu.VMEM((_TAIL, _D), jnp.float32),
            pltpu.VMEM((_TAIL,), jnp.int32),
            pltpu.VMEM((_BLK, _CW), jnp.float32),
            pltpu.VMEM((_RPT, _D), jnp.float32),
            pltpu.VMEM_SHARED((_G, _D), jnp.float32),
            pltpu.VMEM_SHARED((_G, _CW), jnp.float32),
        ],
    )
    def k(feat_h, ids_h, zrows_h, ones_h, sums_o, cnts_o,
          feat_v, ids_v, feat_t, ids_t, ones_v, zf_v, acc, cnt):
        c = lax.axis_index("c")
        s = lax.axis_index("s")
        wid = c * 16 + s

        pltpu.sync_copy(zrows_h, zf_v)
        pltpu.sync_copy(ones_h, ones_v)
        rb = s * _RPT
        pltpu.sync_copy(zf_v, acc.at[pl.ds(rb, _RPT)])
        pltpu.sync_copy(zf_v, cnt.at[pl.ds(rb, _RPT)])
        plsc.subcore_barrier()

        start = wid * _BASE + jnp.minimum(wid, _REM)
        nblk = _BASE + (wid < _REM).astype(jnp.int32)

        def body(i, carry):
            off = (start + i) * _BLK
            pltpu.sync_copy(ids_h.at[pl.ds(off, _BLK)], ids_v)
            pltpu.sync_copy(feat_h.at[pl.ds(off, _BLK)], feat_v)
            pltpu.sync_copy(feat_v, acc.at[ids_v], add=True)
            pltpu.sync_copy(ones_v, cnt.at[ids_v], add=True)
            return carry

        lax.fori_loop(0, nblk, body, 0)

        @pl.when(wid == _NW - 1)
        def _():
            off = _NB_FULL * _BLK
            pltpu.sync_copy(ids_h.at[pl.ds(off, _TAIL)], ids_t)
            pltpu.sync_copy(feat_h.at[pl.ds(off, _TAIL)], feat_t)
            pltpu.sync_copy(feat_t, acc.at[ids_t], add=True)
            pltpu.sync_copy(ones_v.at[pl.ds(0, _TAIL)], cnt.at[ids_t], add=True)

        plsc.subcore_barrier()

        pltpu.sync_copy(acc.at[pl.ds(rb, _RPT)], zf_v)
        pltpu.sync_copy(zf_v, sums_o.at[pl.ds(c * _G + rb, _RPT)])
        pltpu.sync_copy(cnt.at[pl.ds(rb, _RPT)], zf_v)
        pltpu.sync_copy(zf_v, cnts_o.at[pl.ds(c * _G + rb, _RPT)])

    return k(feat, ids, zrows, ones_rows)


def _head(s0, s1, c0, c1, ga, w1a, w1b, b1, gamma, beta, w2t, b2):
    def body(s0_r, s1_r, c0_r, c1_r, ga_r, w1a_r, w1b_r, b1_r, g_r, be_r,
             w2_r, b2_r, o_r):
        ssum = s0_r[...] + s1_r[...]
        cnt = c0_r[...][:, 0:1] + c1_r[...][:, 0:1]
        mean = ssum / jnp.maximum(cnt, 1.0)
        h = (jnp.dot(mean, w1a_r[...], preferred_element_type=jnp.float32)
             + jnp.dot(ga_r[...], w1b_r[...], preferred_element_type=jnp.float32)
             + b1_r[...])
        mu = jnp.mean(h, axis=-1, keepdims=True)
        var = jnp.mean((h - mu) * (h - mu), axis=-1, keepdims=True)
        hn = (h - mu) * lax.rsqrt(var + 1e-5) * g_r[...] + be_r[...]
        r = jnp.maximum(hn, 0.0)
        o_r[...] = jnp.sum(r * w2_r[...], axis=-1, keepdims=True) + b2_r[...]

    return pl.pallas_call(
        body,
        out_shape=jax.ShapeDtypeStruct((_G, 1), jnp.float32),
    )(s0, s1, c0, c1, ga, w1a, w1b, b1, gamma, beta, w2t, b2)


def kernel(node_features, batch, graph_attr, W1, b1, gamma, beta, W2, b2):
    ids = batch.astype(jnp.int32)
    zrows = jnp.zeros((_RPT, _D), jnp.float32)
    ones_rows = jnp.ones((_BLK, _CW), jnp.float32)
    sums, cnts = _segment_sums(node_features, ids, zrows, ones_rows)
    return _head(
        sums[:_G], sums[_G:], cnts[:_G], cnts[_G:], graph_attr,
        W1[:_D], W1[_D:], b1.reshape(1, -1), gamma.reshape(1, -1),
        beta.reshape(1, -1), W2.reshape(1, -1), b2.reshape(1, 1),
    )


# speedup vs baseline: 6.4508x; 1.4198x over previous
import functools

import jax
import jax.numpy as jnp
from jax import lax
from jax.experimental import pallas as pl
from jax.experimental.pallas import tpu as pltpu
from jax.experimental.pallas import tpu_sc as plsc

_G = 512
_N = 100000
_D = 128
_BLK = 128
_NB_FULL = _N // _BLK
_TAIL = _N - _NB_FULL * _BLK
_NW = 32
_CW = 128
_RPT = _G // 16

_SLOTS = 24
_LAST_WID = _NB_FULL - _SLOTS * _NW
_NBUF = 3


def _segment_sums(feat, ids, zrows, ones_rows):
    mesh = plsc.VectorSubcoreMesh(core_axis_name="c", subcore_axis_name="s")

    @functools.partial(
        pl.kernel,
        out_type=[
            jax.ShapeDtypeStruct((2 * _G, _D), jnp.float32),
            jax.ShapeDtypeStruct((2 * _G, _CW), jnp.float32),
        ],
        mesh=mesh,
        scratch_types=[
            [pltpu.VMEM((_BLK, _D), jnp.float32) for _ in range(_NBUF)],
            [pltpu.VMEM((_BLK,), jnp.int32) for _ in range(_NBUF)],
            pltpu.VMEM((_TAIL, _D), jnp.float32),
            pltpu.VMEM((_TAIL,), jnp.int32),
            pltpu.VMEM((_BLK, _CW), jnp.float32),
            pltpu.VMEM((_RPT, _D), jnp.float32),
            pltpu.VMEM_SHARED((_G, _D), jnp.float32),
            pltpu.VMEM_SHARED((_G, _CW), jnp.float32),
            [pltpu.SemaphoreType.DMA for _ in range(_NBUF)],
            [pltpu.SemaphoreType.DMA for _ in range(_NBUF)],
        ],
    )
    def k(feat_h, ids_h, zrows_h, ones_h, sums_o, cnts_o,
          fb, ib, feat_t, ids_t, ones_v, zf_v, acc, cnt, gs, ss):
        c = lax.axis_index("c")
        s = lax.axis_index("s")
        wid = c * 16 + s

        def issue_gather(jj, b):
            off = (jj * _NW + wid) * _BLK
            pltpu.async_copy(ids_h.at[pl.ds(off, _BLK)], ib[b], gs[b])
            pltpu.async_copy(feat_h.at[pl.ds(off, _BLK)], fb[b], gs[b])

        def wait_gather(b):
            pltpu.make_async_copy(ids_h.at[pl.ds(0, _BLK)], ib[b], gs[b]).wait()
            pltpu.make_async_copy(feat_h.at[pl.ds(0, _BLK)], fb[b], gs[b]).wait()

        def issue_scatter(b):
            pltpu.async_copy(fb[b], acc.at[ib[b]], ss[b], add=True)
            pltpu.async_copy(ones_v, cnt.at[ib[b]], ss[b], add=True)

        def wait_scatter(b):
            pltpu.make_async_copy(fb[b], acc.at[ib[b]], ss[b]).wait()
            pltpu.make_async_copy(ones_v, cnt.at[ib[b]], ss[b]).wait()

        issue_gather(0, 0)
        issue_gather(1, 1)

        pltpu.sync_copy(zrows_h, zf_v)
        pltpu.sync_copy(ones_h, ones_v)
        rb = s * _RPT
        pltpu.sync_copy(zf_v, acc.at[pl.ds(rb, _RPT)])
        pltpu.sync_copy(zf_v, cnt.at[pl.ds(rb, _RPT)])
        plsc.subcore_barrier()

        def chunk(i, carry):
            jj0 = i * _NBUF
            for pos in range(_NBUF):
                jj = jj0 + pos
                wait_gather(pos)
                issue_scatter(pos)
                bn = (pos + 2) % _NBUF

                @pl.when(jj >= 1)
                def _():
                    wait_scatter(bn)

                nxt = jj + 2

                @pl.when(nxt < _SLOTS)
                def _():
                    issue_gather(nxt, bn)

                @pl.when((nxt == _SLOTS) & (wid < _LAST_WID))
                def _():
                    issue_gather(_SLOTS, bn)
            return carry

        lax.fori_loop(0, _SLOTS // _NBUF, chunk, 0)

        @pl.when(wid < _LAST_WID)
        def _():
            wait_gather(_SLOTS % _NBUF)
            issue_scatter(_SLOTS % _NBUF)

        @pl.when(wid == _NW - 1)
        def _():
            off = _NB_FULL * _BLK
            pltpu.sync_copy(ids_h.at[pl.ds(off, _TAIL)], ids_t)
            pltpu.sync_copy(feat_h.at[pl.ds(off, _TAIL)], feat_t)
            pltpu.sync_copy(feat_t, acc.at[ids_t], add=True)
            pltpu.sync_copy(ones_v.at[pl.ds(0, _TAIL)], cnt.at[ids_t], add=True)

        wait_scatter((_SLOTS - 1) % _NBUF)

        @pl.when(wid < _LAST_WID)
        def _():
            wait_scatter(_SLOTS % _NBUF)

        plsc.subcore_barrier()

        pltpu.sync_copy(acc.at[pl.ds(rb, _RPT)], zf_v)
        pltpu.sync_copy(zf_v, sums_o.at[pl.ds(c * _G + rb, _RPT)])
        pltpu.sync_copy(cnt.at[pl.ds(rb, _RPT)], zf_v)
        pltpu.sync_copy(zf_v, cnts_o.at[pl.ds(c * _G + rb, _RPT)])

    return k(feat, ids, zrows, ones_rows)


def _head(s0, s1, c0, c1, ga, w1a, w1b, b1, gamma, beta, w2t, b2):
    def body(s0_r, s1_r, c0_r, c1_r, ga_r, w1a_r, w1b_r, b1_r, g_r, be_r,
             w2_r, b2_r, o_r):
        ssum = s0_r[...] + s1_r[...]
        cnt = c0_r[...][:, 0:1] + c1_r[...][:, 0:1]
        mean = ssum / jnp.maximum(cnt, 1.0)
        h = (jnp.dot(mean, w1a_r[...], preferred_element_type=jnp.float32)
             + jnp.dot(ga_r[...], w1b_r[...], preferred_element_type=jnp.float32)
             + b1_r[...])
        mu = jnp.mean(h, axis=-1, keepdims=True)
        var = jnp.mean((h - mu) * (h - mu), axis=-1, keepdims=True)
        hn = (h - mu) * lax.rsqrt(var + 1e-5) * g_r[...] + be_r[...]
        r = jnp.maximum(hn, 0.0)
        o_r[...] = jnp.sum(r * w2_r[...], axis=-1, keepdims=True) + b2_r[...]

    return pl.pallas_call(
        body,
        out_shape=jax.ShapeDtypeStruct((_G, 1), jnp.float32),
    )(s0, s1, c0, c1, ga, w1a, w1b, b1, gamma, beta, w2t, b2)


def kernel(node_features, batch, graph_attr, W1, b1, gamma, beta, W2, b2):
    ids = batch.astype(jnp.int32)
    zrows = jnp.zeros((_RPT, _D), jnp.float32)
    ones_rows = jnp.ones((_BLK, _CW), jnp.float32)
    sums, cnts = _segment_sums(node_features, ids, zrows, ones_rows)
    return _head(
        sums[:_G], sums[_G:], cnts[:_G], cnts[_G:], graph_attr,
        W1[:_D], W1[_D:], b1.reshape(1, -1), gamma.reshape(1, -1),
        beta.reshape(1, -1), W2.reshape(1, -1), b2.reshape(1, 1),
    )


# speedup vs baseline: 6.9529x; 1.0778x over previous
import functools

import jax
import jax.numpy as jnp
from jax import lax
from jax.experimental import pallas as pl
from jax.experimental.pallas import tpu as pltpu
from jax.experimental.pallas import tpu_sc as plsc

_G = 512
_N = 100000
_D = 128
_BLK = 128
_NB_FULL = _N // _BLK
_TAIL = _N - _NB_FULL * _BLK
_NW = 32
_CW = 128
_RPT = _G // 16

_SLOTS = 24
_LAST_WID = _NB_FULL - _SLOTS * _NW
_NBUF = 4


def _segment_sums(feat, ids, zrows, ones_rows):
    mesh = plsc.VectorSubcoreMesh(core_axis_name="c", subcore_axis_name="s")

    @functools.partial(
        pl.kernel,
        out_type=[
            jax.ShapeDtypeStruct((2 * _G, _D), jnp.float32),
            jax.ShapeDtypeStruct((2 * _G, _CW), jnp.float32),
        ],
        mesh=mesh,
        scratch_types=[
            [pltpu.VMEM((_BLK, _D), jnp.float32) for _ in range(_NBUF)],
            [pltpu.VMEM((_BLK,), jnp.int32) for _ in range(_NBUF)],
            pltpu.VMEM((_TAIL, _D), jnp.float32),
            pltpu.VMEM((_TAIL,), jnp.int32),
            pltpu.VMEM((_BLK, _CW), jnp.float32),
            pltpu.VMEM((_RPT, _D), jnp.float32),
            pltpu.VMEM_SHARED((_G, _D), jnp.float32),
            pltpu.VMEM_SHARED((_G, _CW), jnp.float32),
            [pltpu.SemaphoreType.DMA for _ in range(_NBUF)],
            [pltpu.SemaphoreType.DMA for _ in range(_NBUF)],
        ],
    )
    def k(feat_h, ids_h, zrows_h, ones_h, sums_o, cnts_o,
          fb, ib, feat_t, ids_t, ones_v, zf_v, acc, cnt, gs, ss):
        c = lax.axis_index("c")
        s = lax.axis_index("s")
        wid = c * 16 + s

        def issue_gather(jj, b):
            off = (jj * _NW + wid) * _BLK
            pltpu.async_copy(ids_h.at[pl.ds(off, _BLK)], ib[b], gs[b])
            pltpu.async_copy(feat_h.at[pl.ds(off, _BLK)], fb[b], gs[b])

        def wait_gather(b):
            pltpu.make_async_copy(ids_h.at[pl.ds(0, _BLK)], ib[b], gs[b]).wait()
            pltpu.make_async_copy(feat_h.at[pl.ds(0, _BLK)], fb[b], gs[b]).wait()

        def issue_scatter(b):
            pltpu.async_copy(fb[b], acc.at[ib[b]], ss[b], add=True)
            pltpu.async_copy(ones_v, cnt.at[ib[b]], ss[b], add=True)

        def wait_scatter(b):
            pltpu.make_async_copy(fb[b], acc.at[ib[b]], ss[b]).wait()
            pltpu.make_async_copy(ones_v, cnt.at[ib[b]], ss[b]).wait()

        issue_gather(0, 0)
        issue_gather(1, 1)
        issue_gather(2, 2)

        pltpu.sync_copy(zrows_h, zf_v)
        pltpu.sync_copy(ones_h, ones_v)
        rb = s * _RPT
        pltpu.sync_copy(zf_v, acc.at[pl.ds(rb, _RPT)])
        pltpu.sync_copy(zf_v, cnt.at[pl.ds(rb, _RPT)])
        plsc.subcore_barrier()

        def chunk(i, carry):
            jj0 = i * _NBUF
            for pos in range(_NBUF):
                jj = jj0 + pos
                wait_gather(pos)
                issue_scatter(pos)
                bn = (pos + 3) % _NBUF

                @pl.when(jj >= 1)
                def _():
                    wait_scatter(bn)

                nxt = jj + 3

                @pl.when(nxt < _SLOTS)
                def _():
                    issue_gather(nxt, bn)

                @pl.when((nxt == _SLOTS) & (wid < _LAST_WID))
                def _():
                    issue_gather(_SLOTS, bn)
            return carry

        lax.fori_loop(0, _SLOTS // _NBUF, chunk, 0)

        @pl.when(wid < _LAST_WID)
        def _():
            wait_gather(_SLOTS % _NBUF)
            issue_scatter(_SLOTS % _NBUF)

        @pl.when(wid == _NW - 1)
        def _():
            off = _NB_FULL * _BLK
            pltpu.sync_copy(ids_h.at[pl.ds(off, _TAIL)], ids_t)
            pltpu.sync_copy(feat_h.at[pl.ds(off, _TAIL)], feat_t)
            pltpu.sync_copy(feat_t, acc.at[ids_t], add=True)
            pltpu.sync_copy(ones_v.at[pl.ds(0, _TAIL)], cnt.at[ids_t], add=True)

        wait_scatter((_SLOTS - 1) % _NBUF)

        @pl.when(wid < _LAST_WID)
        def _():
            wait_scatter(_SLOTS % _NBUF)

        plsc.subcore_barrier()

        pltpu.sync_copy(acc.at[pl.ds(rb, _RPT)], zf_v)
        pltpu.sync_copy(zf_v, sums_o.at[pl.ds(c * _G + rb, _RPT)])
        pltpu.sync_copy(cnt.at[pl.ds(rb, _RPT)], zf_v)
        pltpu.sync_copy(zf_v, cnts_o.at[pl.ds(c * _G + rb, _RPT)])

    return k(feat, ids, zrows, ones_rows)


def _head(s0, s1, c0, c1, ga, w1a, w1b, b1, gamma, beta, w2t, b2):
    def body(s0_r, s1_r, c0_r, c1_r, ga_r, w1a_r, w1b_r, b1_r, g_r, be_r,
             w2_r, b2_r, o_r):
        ssum = s0_r[...] + s1_r[...]
        cnt = c0_r[...][:, 0:1] + c1_r[...][:, 0:1]
        mean = ssum / jnp.maximum(cnt, 1.0)
        h = (jnp.dot(mean, w1a_r[...], preferred_element_type=jnp.float32)
             + jnp.dot(ga_r[...], w1b_r[...], preferred_element_type=jnp.float32)
             + b1_r[...])
        mu = jnp.mean(h, axis=-1, keepdims=True)
        var = jnp.mean((h - mu) * (h - mu), axis=-1, keepdims=True)
        hn = (h - mu) * lax.rsqrt(var + 1e-5) * g_r[...] + be_r[...]
        r = jnp.maximum(hn, 0.0)
        o_r[...] = jnp.sum(r * w2_r[...], axis=-1, keepdims=True) + b2_r[...]

    return pl.pallas_call(
        body,
        out_shape=jax.ShapeDtypeStruct((_G, 1), jnp.float32),
    )(s0, s1, c0, c1, ga, w1a, w1b, b1, gamma, beta, w2t, b2)


def kernel(node_features, batch, graph_attr, W1, b1, gamma, beta, W2, b2):
    ids = batch.astype(jnp.int32)
    zrows = jnp.zeros((_RPT, _D), jnp.float32)
    ones_rows = jnp.ones((_BLK, _CW), jnp.float32)
    sums, cnts = _segment_sums(node_features, ids, zrows, ones_rows)
    return _head(
        sums[:_G], sums[_G:], cnts[:_G], cnts[_G:], graph_attr,
        W1[:_D], W1[_D:], b1.reshape(1, -1), gamma.reshape(1, -1),
        beta.reshape(1, -1), W2.reshape(1, -1), b2.reshape(1, 1),
    )
